# baseline (speedup 1.0000x reference)
import functools

import jax
import jax.numpy as jnp
from jax import lax
from jax.experimental import pallas as pl
from jax.experimental.pallas import tpu as pltpu
from jax.experimental.pallas import tpu_sc as plsc

VOCAB = 1000000
D_MODEL = 64
BATCH = 4096
HIST = 200

N = BATCH * HIST
NC, NS = 2, 16
NW = NC * NS
PER_W = N // NW
CH = 256
NCH = PER_W // CH
NBUF = 5
LAG = 2
NGROUP = NCH // NBUF

_mesh = plsc.VectorSubcoreMesh(core_axis_name="c", subcore_axis_name="s")


@functools.partial(
    pl.kernel,
    out_type=jax.ShapeDtypeStruct((NW * NCH, CH, D_MODEL), jnp.float32),
    mesh=_mesh,
    scratch_types=[
        pltpu.VMEM((NCH, CH), jnp.int32),
        pltpu.VMEM((NBUF, CH, D_MODEL), jnp.float32),
    ] + [pltpu.SemaphoreType.DMA] * (2 * NBUF),
    compiler_params=pltpu.CompilerParams(use_tc_tiling_on_sc=False),
)
def _embed_sc(x_hbm, tab_hbm, out_hbm, idx_v, rows_v, *sems):
    gsems = sems[:NBUF]
    wsems = sems[NBUF:]
    wid = lax.axis_index("s") * NC + lax.axis_index("c")
    pltpu.sync_copy(x_hbm.at[wid], idx_v)

    def fire_gather(j, b):
        pltpu.async_copy(tab_hbm.at[idx_v.at[j]], rows_v.at[b], gsems[b])

    def wait_gather(b):
        pltpu.make_async_copy(tab_hbm.at[idx_v.at[0]], rows_v.at[b],
                              gsems[b]).wait()

    def wait_writeback(b):
        pltpu.make_async_copy(rows_v.at[b], out_hbm.at[0], wsems[b]).wait()

    for b in range(NBUF):
        fire_gather(b, b)

    def group(g, carry):
        for b in range(NBUF):
            j = g * NBUF + b
            wait_gather(b)
            pltpu.async_copy(rows_v.at[b], out_hbm.at[wid * NCH + j], wsems[b])

            bw = (b - LAG) % NBUF

            @pl.when(jnp.logical_and(j - LAG >= 0, j - LAG + NBUF < NCH))
            def _():
                wait_writeback(bw)
                fire_gather(j - LAG + NBUF, bw)

        return carry

    lax.fori_loop(0, NGROUP, group, 0)

    for b in range(NBUF):
        wait_writeback(b)


def kernel(x, embed_weight):
    xf = x.reshape(-1).astype(jnp.int32).reshape(NW, NCH, CH)
    out = _embed_sc(xf, embed_weight)
    return out.reshape(BATCH, HIST, D_MODEL)


# speedup vs baseline: 1.0017x; 1.0017x over previous
import functools

import jax
import jax.numpy as jnp
from jax import lax
from jax.experimental import pallas as pl
from jax.experimental.pallas import tpu as pltpu
from jax.experimental.pallas import tpu_sc as plsc

VOCAB = 1000000
D_MODEL = 64
BATCH = 4096
HIST = 200

N = BATCH * HIST
NC, NS = 2, 16
NW = NC * NS
PER_W = N // NW
CH = 128
NV = CH // 16
NCH = PER_W // CH
NBUF = 10
LAG = 4
NGROUP = NCH // NBUF

_mesh = plsc.VectorSubcoreMesh(core_axis_name="c", subcore_axis_name="s")


@functools.partial(
    pl.kernel,
    out_type=jax.ShapeDtypeStruct((NW * NCH, CH, D_MODEL), jnp.float32),
    mesh=_mesh,
    scratch_types=[
        pltpu.VMEM((NCH, CH), jnp.int32),
        pltpu.VMEM((NBUF, CH, D_MODEL), jnp.float32),
    ] + [pltpu.SemaphoreType.DMA] * (2 * NBUF),
    compiler_params=pltpu.CompilerParams(use_tc_tiling_on_sc=False),
)
def _embed_sc(x_hbm, tab_hbm, out_hbm, idx_v, rows_v, *sems):
    gsems = sems[:NBUF]
    wsems = sems[NBUF:]
    wid = lax.axis_index("s") * NC + lax.axis_index("c")
    pltpu.sync_copy(x_hbm.at[wid], idx_v)

    def fire_gather(j, b):
        for k in range(NV):
            idx16 = idx_v[j, pl.ds(k * 16, 16)]
            pltpu.async_copy(tab_hbm.at[idx16],
                             rows_v.at[b, pl.ds(k * 16, 16)], gsems[b])

    def wait_gather(b):
        pltpu.make_async_copy(tab_hbm.at[idx_v.at[0]], rows_v.at[b],
                              gsems[b]).wait()

    def wait_writeback(b):
        pltpu.make_async_copy(rows_v.at[b], out_hbm.at[0], wsems[b]).wait()

    for b in range(NBUF):
        fire_gather(b, b)

    def group(g, carry):
        for b in range(NBUF):
            j = g * NBUF + b
            wait_gather(b)
            pltpu.async_copy(rows_v.at[b], out_hbm.at[wid * NCH + j], wsems[b])

            bw = (b - LAG) % NBUF

            @pl.when(jnp.logical_and(j - LAG >= 0, j - LAG + NBUF < NCH))
            def _():
                wait_writeback(bw)
                fire_gather(j - LAG + NBUF, bw)

        return carry

    lax.fori_loop(0, NGROUP, group, 0)

    for b in range(NBUF):
        wait_writeback(b)


def kernel(x, embed_weight):
    xf = x.reshape(-1).astype(jnp.int32).reshape(NW, NCH, CH)
    out = _embed_sc(xf, embed_weight)
    return out.reshape(BATCH, HIST, D_MODEL)
